# baseline (speedup 1.0000x reference)
import jax
import jax.numpy as jnp
from jax import lax
from jax.experimental import pallas as pl
from jax.experimental.pallas import tpu as pltpu

NUM_CH = 8
BN_EPS = 1e-5


def _toeplitz(w_rows, cout, kh, kw, cin, w_pad, wo, w_valid):
    m = jnp.arange(wo * cout)
    k = jnp.arange(kh * w_pad * cin)
    wo_idx = (m // cout)[:, None]
    co = (m % cout)[:, None]
    ih = (k // (w_pad * cin))[None, :]
    w_in = ((k % (w_pad * cin)) // cin)[None, :]
    ci = (k % cin)[None, :]
    dw = w_in - wo_idx
    valid = (dw >= 0) & (dw < kw) & (w_in < w_valid)
    row = (ih * kw + jnp.clip(dw, 0, kw - 1)) * cin + ci
    vals = w_rows[row, jnp.broadcast_to(co, row.shape)]
    return jnp.where(valid, vals, 0.0).astype(jnp.bfloat16)


def _bn_scale_shift(s, s2, count, gamma, beta):
    n = s.shape[1]
    inv = 1.0 / float(count)
    scs, shs = [], []
    for h in range(n // 128):
        sl = slice(128 * h, 128 * (h + 1))
        mean = jnp.sum(s[:, sl], axis=1, keepdims=True) * inv
        ex2 = jnp.sum(s2[:, sl], axis=1, keepdims=True) * inv
        var = ex2 - mean * mean
        sc = lax.rsqrt(var + BN_EPS) * gamma
        sh = beta - mean * sc
        scs.append(jnp.broadcast_to(sc, (s.shape[0], 128)))
        shs.append(jnp.broadcast_to(sh, (s.shape[0], 128)))
    if len(scs) == 1:
        return scs[0], shs[0]
    return jnp.concatenate(scs, axis=1), jnp.concatenate(shs, axis=1)


def _model_kernel(x_ref,
                  t1_ref, cb1_ref, g1_ref, be1_ref,
                  t2_ref, cb2_ref,
                  t3_ref, cb3_ref, g3_ref, be3_ref,
                  fw1_ref, fb1_ref, fw2_ref, fb2_ref,
                  o_ref,
                  xs_ref, a1_ref, a1p_ref, a2_ref, a2b_ref):
    N = x_ref.shape[0]
    f32 = jnp.float32
    bf16 = jnp.bfloat16

    def dot(t_ref, slab):
        return jnp.dot(t_ref[...], slab, preferred_element_type=f32)

    xs_ref[...] = jnp.transpose(x_ref[...])

    s1 = jnp.zeros((NUM_CH, N), f32)
    s1q = jnp.zeros((NUM_CH, N), f32)
    for p in range(12):
        r0 = 2 * p
        d0 = dot(t1_ref, xs_ref[32 * r0:32 * r0 + 160])
        d1 = dot(t1_ref, xs_ref[32 * r0 + 32:32 * r0 + 192])
        m = jnp.maximum(d0, d1).reshape(12, 2, NUM_CH, N)
        pooled = jnp.maximum(m[:, 0], m[:, 1]) + cb1_ref[...]
        a1_ref[p] = pooled
        s1 = s1 + jnp.sum(pooled, axis=0)
        s1q = s1q + jnp.sum(pooled * pooled, axis=0)
    sc1, sh1 = _bn_scale_shift(s1, s1q, 12 * 12 * 128, g1_ref[...], be1_ref[...])

    a1p_ref[...] = jnp.zeros(a1p_ref.shape, bf16)
    a1p_ref[1:13, 1:13] = jnp.maximum(
        a1_ref[...] * sc1 + sh1, 0.0).astype(bf16)

    s2 = jnp.zeros((NUM_CH, N), f32)
    s2q = jnp.zeros((NUM_CH, N), f32)
    for r in range(12):
        d = dot(t2_ref, a1p_ref[r:r + 3].reshape(336, N))
        row = d.reshape(12, NUM_CH, N) + cb2_ref[...]
        a2_ref[r] = row
        s2 = s2 + jnp.sum(row, axis=0)
        s2q = s2q + jnp.sum(row * row, axis=0)
    sc2, sh2 = _bn_scale_shift(s2, s2q, 12 * 12 * 128, g1_ref[...], be1_ref[...])
    a2b_ref[...] = jnp.maximum(a2_ref[...] * sc2 + sh2, 0.0).astype(bf16)

    s3 = jnp.zeros((NUM_CH, N), f32)
    s3q = jnp.zeros((NUM_CH, N), f32)
    a3_rows = []
    for p in range(4):
        r0 = 2 * p
        d0 = dot(t3_ref, a2b_ref[r0:r0 + 5].reshape(480, N))
        d1 = dot(t3_ref, a2b_ref[r0 + 1:r0 + 6].reshape(480, N))
        m = jnp.maximum(d0, d1).reshape(4, 2, NUM_CH, N)
        pooled = jnp.maximum(m[:, 0], m[:, 1]) + cb3_ref[...]
        a3_rows.append(pooled)
        s3 = s3 + jnp.sum(pooled, axis=0)
        s3q = s3q + jnp.sum(pooled * pooled, axis=0)
    sc3, sh3 = _bn_scale_shift(s3, s3q, 4 * 4 * 128, g3_ref[...], be3_ref[...])

    a3 = jnp.concatenate(a3_rows, axis=0).reshape(16, NUM_CH, N)
    feat = jnp.maximum(a3 * sc3 + sh3, 0.0).reshape(4 * 4 * NUM_CH, N)

    h = jnp.maximum(
        jnp.dot(fw1_ref[...], feat, preferred_element_type=f32) + fb1_ref[...],
        0.0)
    z = jnp.dot(fw2_ref[...], h, preferred_element_type=f32) + fb2_ref[...]
    o_ref[...] = jnp.maximum(z, 0.0)


def kernel(x, w1, cb1, g1, be1, w2, cb2, w3, cb3, g3, be3,
           fc1_w, fc1_b, fc2_w, fc2_b):
    B = x.shape[0]
    bt = 256 if B % 256 == 0 else 128
    assert B % bt == 0

    x_pad = jnp.pad(x.reshape(B, 28, 28),
                    ((0, 0), (0, 0), (0, 4))).astype(jnp.bfloat16)
    x_pad = x_pad.reshape(B, 896)

    t1 = _toeplitz(w1, NUM_CH, 5, 5, 1, 32, 24, 28)
    t2 = _toeplitz(w2, NUM_CH, 3, 3, NUM_CH, 14, 12, 14)
    t3 = _toeplitz(w3, NUM_CH, 5, 5, NUM_CH, 12, 8, 12)

    def full(arr):
        nd = arr.ndim
        return pl.BlockSpec(arr.shape, lambda b, _nd=nd: (0,) * _nd)

    in_specs = [
        pl.BlockSpec((bt, 896), lambda b: (b, 0)),
        full(t1), full(cb1), full(g1), full(be1),
        full(t2), full(cb2),
        full(t3), full(cb3), full(g3), full(be3),
        full(fc1_w), full(fc1_b), full(fc2_w), full(fc2_b),
    ]

    out = pl.pallas_call(
        _model_kernel,
        out_shape=jax.ShapeDtypeStruct((10, B), jnp.float32),
        grid_spec=pltpu.PrefetchScalarGridSpec(
            num_scalar_prefetch=0,
            grid=(B // bt,),
            in_specs=in_specs,
            out_specs=pl.BlockSpec((10, bt), lambda b: (0, b)),
            scratch_shapes=[
                pltpu.VMEM((896, bt), jnp.bfloat16),
                pltpu.VMEM((12, 12, NUM_CH, bt), jnp.float32),
                pltpu.VMEM((14, 14, NUM_CH, bt), jnp.bfloat16),
                pltpu.VMEM((12, 12, NUM_CH, bt), jnp.float32),
                pltpu.VMEM((12, 12, NUM_CH, bt), jnp.bfloat16),
            ]),
        compiler_params=pltpu.CompilerParams(
            dimension_semantics=("parallel",),
            vmem_limit_bytes=64 * 1024 * 1024),
    )(x_pad, t1, cb1, g1, be1, t2, cb2, t3, cb3, g3, be3,
      fc1_w, fc1_b, fc2_w, fc2_b)

    return out.T


# speedup vs baseline: 4.6478x; 4.6478x over previous
import jax
import jax.numpy as jnp
from jax import lax
from jax.experimental import pallas as pl
from jax.experimental.pallas import tpu as pltpu

NUM_CH = 8
BN_EPS = 1e-5


def _toeplitz(w_rows, cout, kh, kw, cin, w_pad, wo, w_valid):
    period = (w_valid + 1) * cin
    band = kw * cin
    u = w_rows.reshape(kh, kw, cin, cout).transpose(0, 3, 1, 2).reshape(kh, cout, band)
    buf = jnp.concatenate([u, jnp.zeros((kh, cout, period - band), u.dtype)], axis=2)
    buf = jnp.tile(buf, (1, 1, wo))[:, :, :wo * w_valid * cin]
    t = buf.reshape(kh, cout, wo, w_valid, cin)
    if w_pad != w_valid:
        t = jnp.pad(t, ((0, 0), (0, 0), (0, 0), (0, w_pad - w_valid), (0, 0)))
    t = t.reshape(kh, cout, wo, w_pad * cin).transpose(2, 1, 0, 3)
    return t.reshape(wo * cout, kh * w_pad * cin).astype(jnp.bfloat16)


def _bn_scale_shift(s, s2, count, gamma, beta):
    n = s.shape[1]
    inv = 1.0 / float(count)
    scs, shs = [], []
    for h in range(n // 128):
        sl = slice(128 * h, 128 * (h + 1))
        mean = jnp.sum(s[:, sl], axis=1, keepdims=True) * inv
        ex2 = jnp.sum(s2[:, sl], axis=1, keepdims=True) * inv
        var = ex2 - mean * mean
        sc = lax.rsqrt(var + BN_EPS) * gamma
        sh = beta - mean * sc
        scs.append(jnp.broadcast_to(sc, (s.shape[0], 128)))
        shs.append(jnp.broadcast_to(sh, (s.shape[0], 128)))
    if len(scs) == 1:
        return scs[0], shs[0]
    return jnp.concatenate(scs, axis=1), jnp.concatenate(shs, axis=1)


def _model_kernel(x_ref,
                  t1_ref, cb1_ref, g1_ref, be1_ref,
                  t2_ref, cb2_ref,
                  t3_ref, cb3_ref, g3_ref, be3_ref,
                  fw1_ref, fb1_ref, fw2_ref, fb2_ref,
                  o_ref,
                  xs_ref, a1_ref, a1p_ref, a2_ref, a2b_ref):
    N = x_ref.shape[0]
    f32 = jnp.float32
    bf16 = jnp.bfloat16

    def dot(t_ref, slab):
        return jnp.dot(t_ref[...], slab, preferred_element_type=f32)

    xs_ref[...] = jnp.transpose(x_ref[...])

    s1 = jnp.zeros((NUM_CH, N), f32)
    s1q = jnp.zeros((NUM_CH, N), f32)
    for p in range(12):
        r0 = 2 * p
        d0 = dot(t1_ref, xs_ref[32 * r0:32 * r0 + 160])
        d1 = dot(t1_ref, xs_ref[32 * r0 + 32:32 * r0 + 192])
        m = jnp.maximum(d0, d1).reshape(12, 2, NUM_CH, N)
        pooled = jnp.maximum(m[:, 0], m[:, 1]) + cb1_ref[...]
        a1_ref[p] = pooled
        s1 = s1 + jnp.sum(pooled, axis=0)
        s1q = s1q + jnp.sum(pooled * pooled, axis=0)
    sc1, sh1 = _bn_scale_shift(s1, s1q, 12 * 12 * 128, g1_ref[...], be1_ref[...])

    a1p_ref[...] = jnp.zeros(a1p_ref.shape, bf16)
    a1p_ref[1:13, 1:13] = jnp.maximum(
        a1_ref[...] * sc1 + sh1, 0.0).astype(bf16)

    s2 = jnp.zeros((NUM_CH, N), f32)
    s2q = jnp.zeros((NUM_CH, N), f32)
    for r in range(12):
        d = dot(t2_ref, a1p_ref[r:r + 3].reshape(336, N))
        row = d.reshape(12, NUM_CH, N) + cb2_ref[...]
        a2_ref[r] = row
        s2 = s2 + jnp.sum(row, axis=0)
        s2q = s2q + jnp.sum(row * row, axis=0)
    sc2, sh2 = _bn_scale_shift(s2, s2q, 12 * 12 * 128, g1_ref[...], be1_ref[...])
    a2b_ref[...] = jnp.maximum(a2_ref[...] * sc2 + sh2, 0.0).astype(bf16)

    s3 = jnp.zeros((NUM_CH, N), f32)
    s3q = jnp.zeros((NUM_CH, N), f32)
    a3_rows = []
    for p in range(4):
        r0 = 2 * p
        d0 = dot(t3_ref, a2b_ref[r0:r0 + 5].reshape(480, N))
        d1 = dot(t3_ref, a2b_ref[r0 + 1:r0 + 6].reshape(480, N))
        m = jnp.maximum(d0, d1).reshape(4, 2, NUM_CH, N)
        pooled = jnp.maximum(m[:, 0], m[:, 1]) + cb3_ref[...]
        a3_rows.append(pooled)
        s3 = s3 + jnp.sum(pooled, axis=0)
        s3q = s3q + jnp.sum(pooled * pooled, axis=0)
    sc3, sh3 = _bn_scale_shift(s3, s3q, 4 * 4 * 128, g3_ref[...], be3_ref[...])

    a3 = jnp.concatenate(a3_rows, axis=0).reshape(16, NUM_CH, N)
    feat = jnp.maximum(a3 * sc3 + sh3, 0.0).reshape(4 * 4 * NUM_CH, N)

    h = jnp.maximum(
        jnp.dot(fw1_ref[...], feat, preferred_element_type=f32) + fb1_ref[...],
        0.0)
    z = jnp.dot(fw2_ref[...], h, preferred_element_type=f32) + fb2_ref[...]
    o_ref[...] = jnp.maximum(z, 0.0)


def kernel(x, w1, cb1, g1, be1, w2, cb2, w3, cb3, g3, be3,
           fc1_w, fc1_b, fc2_w, fc2_b):
    B = x.shape[0]
    bt = 256 if B % 256 == 0 else 128
    assert B % bt == 0

    x_pad = jnp.pad(x.reshape(B, 28, 28),
                    ((0, 0), (0, 0), (0, 4))).astype(jnp.bfloat16)
    x_pad = x_pad.reshape(B, 896)

    t1 = _toeplitz(w1, NUM_CH, 5, 5, 1, 32, 24, 28)
    t2 = _toeplitz(w2, NUM_CH, 3, 3, NUM_CH, 14, 12, 14)
    t3 = _toeplitz(w3, NUM_CH, 5, 5, NUM_CH, 12, 8, 12)

    def full(arr):
        nd = arr.ndim
        return pl.BlockSpec(arr.shape, lambda b, _nd=nd: (0,) * _nd)

    in_specs = [
        pl.BlockSpec((bt, 896), lambda b: (b, 0)),
        full(t1), full(cb1), full(g1), full(be1),
        full(t2), full(cb2),
        full(t3), full(cb3), full(g3), full(be3),
        full(fc1_w), full(fc1_b), full(fc2_w), full(fc2_b),
    ]

    out = pl.pallas_call(
        _model_kernel,
        out_shape=jax.ShapeDtypeStruct((10, B), jnp.float32),
        grid_spec=pltpu.PrefetchScalarGridSpec(
            num_scalar_prefetch=0,
            grid=(B // bt,),
            in_specs=in_specs,
            out_specs=pl.BlockSpec((10, bt), lambda b: (0, b)),
            scratch_shapes=[
                pltpu.VMEM((896, bt), jnp.bfloat16),
                pltpu.VMEM((12, 12, NUM_CH, bt), jnp.float32),
                pltpu.VMEM((14, 14, NUM_CH, bt), jnp.bfloat16),
                pltpu.VMEM((12, 12, NUM_CH, bt), jnp.float32),
                pltpu.VMEM((12, 12, NUM_CH, bt), jnp.bfloat16),
            ]),
        compiler_params=pltpu.CompilerParams(
            dimension_semantics=("parallel",),
            vmem_limit_bytes=64 * 1024 * 1024),
    )(x_pad, t1, cb1, g1, be1, t2, cb2, t3, cb3, g3, be3,
      fc1_w, fc1_b, fc2_w, fc2_b)

    return out.T
